# baseline (speedup 1.0000x reference)
import functools

import jax
import jax.numpy as jnp
from jax import lax
from jax.experimental import pallas as pl
from jax.experimental.pallas import tpu as pltpu
from jax.experimental.pallas import tpu_sc as plsc

N_NODES = 10000
DEG = 32
D = 128
LANES = 16
REGS = D // LANES

NC, NS = 2, 16
NW = NC * NS
NPW = 320
NPAD = NW * NPW
CHUNK_NODES = 4
CHUNK_EDGES = CHUNK_NODES * DEG
CHUNKS = NPW // CHUNK_NODES
EPW = NPW * DEG
NBUF = 2
SROWS = 624


def _sc_segment_gather_sum(y, idx3):
  mesh = plsc.VectorSubcoreMesh(core_axis_name="c", subcore_axis_name="s")

  @functools.partial(
      pl.kernel,
      out_type=jax.ShapeDtypeStruct((NPAD, D), jnp.float32),
      mesh=mesh,
      scratch_types=[
          pltpu.VMEM((CHUNKS, CHUNK_EDGES), jnp.int32),
          pltpu.VMEM((NBUF, CHUNK_EDGES, D), jnp.float32),
          pltpu.VMEM((NBUF, CHUNK_NODES, D), jnp.float32),
          pltpu.VMEM_SHARED((N_NODES, D), jnp.float32),
          [pltpu.SemaphoreType.DMA] * NBUF,
          [pltpu.SemaphoreType.DMA] * NBUF,
      ],
  )
  def body(y_hbm, idx_hbm, out_hbm, idx_v, gbuf, sbuf, y_sp, gsems, ssems):
    wid = lax.axis_index("s") * NC + lax.axis_index("c")
    sid = lax.axis_index("s")
    pltpu.sync_copy(y_hbm.at[pl.ds(sid * SROWS, SROWS)],
                    y_sp.at[pl.ds(sid * SROWS, SROWS)])

    @pl.when(sid == 0)
    def _():
      pltpu.sync_copy(y_hbm.at[pl.ds(NS * SROWS, N_NODES - NS * SROWS)],
                      y_sp.at[pl.ds(NS * SROWS, N_NODES - NS * SROWS)])
    pltpu.sync_copy(idx_hbm.at[wid], idx_v)
    plsc.subcore_barrier()

    def start(chunk, slot):
      pltpu.async_copy(y_sp.at[idx_v.at[chunk]], gbuf.at[slot], gsems[slot])

    def wait(slot):
      pltpu.make_async_copy(y_sp.at[idx_v.at[0]], gbuf.at[slot],
                            gsems[slot]).wait()

    def store(chunk, slot):
      pltpu.async_copy(
          sbuf.at[slot],
          out_hbm.at[pl.ds(wid * NPW + chunk * CHUNK_NODES, CHUNK_NODES)],
          ssems[slot])

    def store_wait(slot):
      pltpu.make_async_copy(
          sbuf.at[slot], out_hbm.at[pl.ds(0, CHUNK_NODES)], ssems[slot]).wait()

    def process(chunk, slot):
      for n in range(CHUNK_NODES):
        base = n * DEG

        def edge_body(e8, accs):
          out = list(accs)
          for de in range(8):
            row = base + e8 * 8 + de
            for r in range(REGS):
              out[r] = out[r] + gbuf[slot, row, pl.ds(r * LANES, LANES)]
          return tuple(out)

        zeros = tuple(jnp.zeros((LANES,), jnp.float32) for _ in range(REGS))
        accs = lax.fori_loop(0, DEG // 8, edge_body, zeros)
        for r in range(REGS):
          sbuf[slot, n, pl.ds(r * LANES, LANES)] = accs[r]
      store(chunk, slot)

    for slot in range(NBUF - 1):
      start(slot, slot)

    def loop_body(g2, carry):
      for b in range(NBUF):
        c = NBUF * g2 + b

        @pl.when(c + NBUF - 1 < CHUNKS)
        def _():
          start(c + NBUF - 1, (b + NBUF - 1) % NBUF)

        wait(b)

        @pl.when(c >= NBUF)
        def _():
          store_wait(b)
        process(c, b)
      return carry

    lax.fori_loop(0, CHUNKS // NBUF, loop_body, 0)
    for slot in range(NBUF):
      store_wait(slot)

  return body(y, idx3)


def _tc_fused_kernel(g_ref, y_ref, w1_ref, w2_ref, b_ref, out_ref):
  acc = jnp.dot(g_ref[...], w1_ref[...],
                preferred_element_type=jnp.float32,
                precision=lax.Precision.HIGHEST)
  acc += jnp.dot(y_ref[...] * jnp.float32(DEG), w2_ref[...],
                 preferred_element_type=jnp.float32,
                 precision=lax.Precision.HIGHEST)
  out_ref[...] = acc + jnp.float32(DEG) * b_ref[...]


def _tc_fused(g_pad, y, w1, w2, b2d):
  blk = 1000
  return pl.pallas_call(
      _tc_fused_kernel,
      out_shape=jax.ShapeDtypeStruct((N_NODES, D), jnp.float32),
      grid=(N_NODES // blk,),
      in_specs=[
          pl.BlockSpec((blk, D), lambda i: (i, 0)),
          pl.BlockSpec((blk, D), lambda i: (i, 0)),
          pl.BlockSpec((D, D), lambda i: (0, 0)),
          pl.BlockSpec((D, D), lambda i: (0, 0)),
          pl.BlockSpec((1, D), lambda i: (0, 0)),
      ],
      out_specs=pl.BlockSpec((blk, D), lambda i: (i, 0)),
  )(g_pad, y, w1, w2, b2d)


@jax.jit
def kernel(y, neighbors_index, neighbors_row_splits, W, b):
  del neighbors_row_splits
  n_edges = neighbors_index.shape[0]
  pad = jnp.arange(NW * EPW - n_edges, dtype=jnp.int32) % N_NODES
  idx3 = jnp.concatenate(
      [neighbors_index.reshape(n_edges // CHUNK_EDGES, CHUNK_EDGES),
       pad.reshape(-1, CHUNK_EDGES)]).reshape(NW, CHUNKS, CHUNK_EDGES)
  g_pad = _sc_segment_gather_sum(y, idx3)
  return _tc_fused(g_pad, y, W[:D], W[D:], b.reshape(1, D))


# speedup vs baseline: 1.0571x; 1.0571x over previous
import functools

import jax
import jax.numpy as jnp
from jax import lax
from jax.experimental import pallas as pl
from jax.experimental.pallas import tpu as pltpu
from jax.experimental.pallas import tpu_sc as plsc

N_NODES = 10000
DEG = 32
D = 128
LANES = 16
REGS = D // LANES

NC, NS = 2, 16
NW = NC * NS
NPW = 320
NPAD = NW * NPW
CHUNK_NODES = 4
CHUNK_EDGES = CHUNK_NODES * DEG
CHUNKS = NPW // CHUNK_NODES
EPW = NPW * DEG
NBUF = 2
SROWS = 624


def _sc_segment_gather_sum(y, idx3):
  mesh = plsc.VectorSubcoreMesh(core_axis_name="c", subcore_axis_name="s")

  @functools.partial(
      pl.kernel,
      out_type=jax.ShapeDtypeStruct((NPAD, D), jnp.float32),
      mesh=mesh,
      scratch_types=[
          pltpu.VMEM((CHUNKS, CHUNK_EDGES), jnp.int32),
          pltpu.VMEM((NBUF, CHUNK_EDGES, D), jnp.float32),
          pltpu.VMEM((NBUF, CHUNK_NODES, D), jnp.float32),
          pltpu.VMEM_SHARED((N_NODES, D), jnp.float32),
          [pltpu.SemaphoreType.DMA] * NBUF,
          [pltpu.SemaphoreType.DMA] * NBUF,
      ],
  )
  def body(y_hbm, idx_hbm, out_hbm, idx_v, gbuf, sbuf, y_sp, gsems, ssems):
    wid = lax.axis_index("s") * NC + lax.axis_index("c")
    sid = lax.axis_index("s")
    pltpu.sync_copy(y_hbm.at[pl.ds(sid * SROWS, SROWS)],
                    y_sp.at[pl.ds(sid * SROWS, SROWS)])

    @pl.when(sid == 0)
    def _():
      pltpu.sync_copy(y_hbm.at[pl.ds(NS * SROWS, N_NODES - NS * SROWS)],
                      y_sp.at[pl.ds(NS * SROWS, N_NODES - NS * SROWS)])
    pltpu.sync_copy(idx_hbm.at[wid], idx_v)
    plsc.subcore_barrier()

    def start(chunk, slot):
      pltpu.async_copy(y_sp.at[idx_v.at[chunk]], gbuf.at[slot], gsems[slot])

    def wait(slot):
      pltpu.make_async_copy(y_sp.at[idx_v.at[0]], gbuf.at[slot],
                            gsems[slot]).wait()

    def store(chunk, slot):
      pltpu.async_copy(
          sbuf.at[slot],
          out_hbm.at[pl.ds(wid * NPW + chunk * CHUNK_NODES, CHUNK_NODES)],
          ssems[slot])

    def store_wait(slot):
      pltpu.make_async_copy(
          sbuf.at[slot], out_hbm.at[pl.ds(0, CHUNK_NODES)], ssems[slot]).wait()

    def process(chunk, slot):
      for n in range(CHUNK_NODES):
        base = n * DEG

        def edge_body(e8, accs):
          out = list(accs)
          for de in range(8):
            row = base + e8 * 8 + de
            for r in range(REGS):
              out[r] = out[r] + gbuf[slot, row, pl.ds(r * LANES, LANES)]
          return tuple(out)

        zeros = tuple(jnp.zeros((LANES,), jnp.float32) for _ in range(REGS))
        accs = lax.fori_loop(0, DEG // 8, edge_body, zeros)
        for r in range(REGS):
          sbuf[slot, n, pl.ds(r * LANES, LANES)] = accs[r]
      store(chunk, slot)

    for slot in range(NBUF - 1):
      start(slot, slot)

    def loop_body(g2, carry):
      for b in range(NBUF):
        c = NBUF * g2 + b

        @pl.when(c + NBUF - 1 < CHUNKS)
        def _():
          start(c + NBUF - 1, (b + NBUF - 1) % NBUF)

        wait(b)

        @pl.when(c >= NBUF)
        def _():
          store_wait(b)
        process(c, b)
      return carry

    lax.fori_loop(0, CHUNKS // NBUF, loop_body, 0)
    for slot in range(NBUF):
      store_wait(slot)

  return body(y, idx3)


def _tc_partial_kernel(y_ref, w2_ref, b_ref, out_ref):
  acc = jnp.dot(y_ref[...] * jnp.float32(DEG), w2_ref[...],
                preferred_element_type=jnp.float32,
                precision=lax.Precision.HIGHEST)
  out_ref[...] = acc + jnp.float32(DEG) * b_ref[...]


def _tc_partial(y, w2, b2d):
  blk = 2000
  return pl.pallas_call(
      _tc_partial_kernel,
      out_shape=jax.ShapeDtypeStruct((N_NODES, D), jnp.float32),
      grid=(N_NODES // blk,),
      in_specs=[
          pl.BlockSpec((blk, D), lambda i: (i, 0)),
          pl.BlockSpec((D, D), lambda i: (0, 0)),
          pl.BlockSpec((1, D), lambda i: (0, 0)),
      ],
      out_specs=pl.BlockSpec((blk, D), lambda i: (i, 0)),
  )(y, w2, b2d)


def _tc_final_kernel(g_ref, w1_ref, p_ref, out_ref):
  out_ref[...] = p_ref[...] + jnp.dot(g_ref[...], w1_ref[...],
                                      preferred_element_type=jnp.float32,
                                      precision=lax.Precision.HIGHEST)


def _tc_final(g_pad, w1, p):
  blk = 1000
  return pl.pallas_call(
      _tc_final_kernel,
      out_shape=jax.ShapeDtypeStruct((N_NODES, D), jnp.float32),
      grid=(N_NODES // blk,),
      in_specs=[
          pl.BlockSpec((blk, D), lambda i: (i, 0)),
          pl.BlockSpec((D, D), lambda i: (0, 0)),
          pl.BlockSpec((blk, D), lambda i: (i, 0)),
      ],
      out_specs=pl.BlockSpec((blk, D), lambda i: (i, 0)),
  )(g_pad, w1, p)


@jax.jit
def kernel(y, neighbors_index, neighbors_row_splits, W, b):
  del neighbors_row_splits
  n_edges = neighbors_index.shape[0]
  pad = jnp.arange(NW * EPW - n_edges, dtype=jnp.int32) % N_NODES
  idx3 = jnp.concatenate(
      [neighbors_index.reshape(n_edges // CHUNK_EDGES, CHUNK_EDGES),
       pad.reshape(-1, CHUNK_EDGES)]).reshape(NW, CHUNKS, CHUNK_EDGES)
  p = _tc_partial(y, W[D:], b.reshape(1, D))
  g_pad = _sc_segment_gather_sum(y, idx3)
  return _tc_final(g_pad, W[:D], p)


# speedup vs baseline: 1.0675x; 1.0099x over previous
import functools

import jax
import jax.numpy as jnp
from jax import lax
from jax.experimental import pallas as pl
from jax.experimental.pallas import tpu as pltpu
from jax.experimental.pallas import tpu_sc as plsc

N_NODES = 10000
DEG = 32
D = 128
LANES = 16
REGS = D // LANES

NC, NS = 2, 16
NW = NC * NS
NPW = 320
NPAD = NW * NPW
CHUNK_NODES = 4
CHUNK_EDGES = CHUNK_NODES * DEG
CHUNKS = NPW // CHUNK_NODES
EPW = NPW * DEG
NBUF = 2
SROWS = 624


def _sc_segment_gather_sum(y, idx3):
  mesh = plsc.VectorSubcoreMesh(core_axis_name="c", subcore_axis_name="s")

  @functools.partial(
      pl.kernel,
      out_type=jax.ShapeDtypeStruct((NPAD, D), jnp.float32),
      mesh=mesh,
      scratch_types=[
          pltpu.VMEM((CHUNKS, CHUNK_EDGES), jnp.int32),
          pltpu.VMEM((NBUF, CHUNK_EDGES, D), jnp.float32),
          pltpu.VMEM((NBUF, CHUNK_NODES, D), jnp.float32),
          pltpu.VMEM_SHARED((N_NODES, D), jnp.float32),
          [pltpu.SemaphoreType.DMA] * NBUF,
          [pltpu.SemaphoreType.DMA] * NBUF,
      ],
  )
  def body(y_hbm, idx_hbm, out_hbm, idx_v, gbuf, sbuf, y_sp, gsems, ssems):
    wid = lax.axis_index("s") * NC + lax.axis_index("c")
    sid = lax.axis_index("s")
    pltpu.sync_copy(y_hbm.at[pl.ds(sid * SROWS, SROWS)],
                    y_sp.at[pl.ds(sid * SROWS, SROWS)])

    @pl.when(sid == 0)
    def _():
      pltpu.sync_copy(y_hbm.at[pl.ds(NS * SROWS, N_NODES - NS * SROWS)],
                      y_sp.at[pl.ds(NS * SROWS, N_NODES - NS * SROWS)])
    pltpu.sync_copy(idx_hbm.at[wid], idx_v)
    plsc.subcore_barrier()

    def start(chunk, slot):
      pltpu.async_copy(y_sp.at[idx_v.at[chunk]], gbuf.at[slot], gsems[slot])

    def wait(slot):
      pltpu.make_async_copy(y_sp.at[idx_v.at[0]], gbuf.at[slot],
                            gsems[slot]).wait()

    def store(chunk, slot):
      pltpu.async_copy(
          sbuf.at[slot],
          out_hbm.at[pl.ds(wid * NPW + chunk * CHUNK_NODES, CHUNK_NODES)],
          ssems[slot])

    def store_wait(slot):
      pltpu.make_async_copy(
          sbuf.at[slot], out_hbm.at[pl.ds(0, CHUNK_NODES)], ssems[slot]).wait()

    def process(chunk, slot):
      for n in range(CHUNK_NODES):
        base = n * DEG

        def edge_body(e8, accs):
          out = list(accs)
          for de in range(4):
            row = base + e8 * 4 + de
            for r in range(REGS):
              out[r] = out[r] + gbuf[slot, row, pl.ds(r * LANES, LANES)]
          return tuple(out)

        zeros = tuple(jnp.zeros((LANES,), jnp.float32) for _ in range(REGS))
        accs = lax.fori_loop(0, DEG // 4, edge_body, zeros)
        for r in range(REGS):
          sbuf[slot, n, pl.ds(r * LANES, LANES)] = accs[r]
      store(chunk, slot)

    for slot in range(NBUF - 1):
      start(slot, slot)

    def loop_body(g2, carry):
      for b in range(NBUF):
        c = NBUF * g2 + b

        @pl.when(c + NBUF - 1 < CHUNKS)
        def _():
          start(c + NBUF - 1, (b + NBUF - 1) % NBUF)

        wait(b)

        @pl.when(c >= NBUF)
        def _():
          store_wait(b)
        process(c, b)
      return carry

    lax.fori_loop(0, CHUNKS // NBUF, loop_body, 0)
    for slot in range(NBUF):
      store_wait(slot)

  return body(y, idx3)


def _tc_partial_kernel(y_ref, w2_ref, b_ref, out_ref):
  acc = jnp.dot(y_ref[...] * jnp.float32(DEG), w2_ref[...],
                preferred_element_type=jnp.float32,
                precision=lax.Precision.HIGHEST)
  out_ref[...] = acc + jnp.float32(DEG) * b_ref[...]


def _tc_partial(y, w2, b2d):
  blk = 2000
  return pl.pallas_call(
      _tc_partial_kernel,
      out_shape=jax.ShapeDtypeStruct((N_NODES, D), jnp.float32),
      grid=(N_NODES // blk,),
      in_specs=[
          pl.BlockSpec((blk, D), lambda i: (i, 0)),
          pl.BlockSpec((D, D), lambda i: (0, 0)),
          pl.BlockSpec((1, D), lambda i: (0, 0)),
      ],
      out_specs=pl.BlockSpec((blk, D), lambda i: (i, 0)),
      compiler_params=pltpu.CompilerParams(
          dimension_semantics=("parallel",)),
  )(y, w2, b2d)


def _tc_final_kernel(g_ref, w1_ref, p_ref, out_ref):
  out_ref[...] = p_ref[...] + jnp.dot(g_ref[...], w1_ref[...],
                                      preferred_element_type=jnp.float32,
                                      precision=lax.Precision.HIGHEST)


def _tc_final(g_pad, w1, p):
  blk = 1000
  return pl.pallas_call(
      _tc_final_kernel,
      out_shape=jax.ShapeDtypeStruct((N_NODES, D), jnp.float32),
      grid=(N_NODES // blk,),
      in_specs=[
          pl.BlockSpec((blk, D), lambda i: (i, 0)),
          pl.BlockSpec((D, D), lambda i: (0, 0)),
          pl.BlockSpec((blk, D), lambda i: (i, 0)),
      ],
      out_specs=pl.BlockSpec((blk, D), lambda i: (i, 0)),
      compiler_params=pltpu.CompilerParams(
          dimension_semantics=("parallel",)),
  )(g_pad, w1, p)


@jax.jit
def kernel(y, neighbors_index, neighbors_row_splits, W, b):
  del neighbors_row_splits
  n_edges = neighbors_index.shape[0]
  pad = jnp.arange(NW * EPW - n_edges, dtype=jnp.int32) % N_NODES
  idx3 = jnp.concatenate(
      [neighbors_index.reshape(n_edges // CHUNK_EDGES, CHUNK_EDGES),
       pad.reshape(-1, CHUNK_EDGES)]).reshape(NW, CHUNKS, CHUNK_EDGES)
  p = _tc_partial(y, W[D:], b.reshape(1, D))
  g_pad = _sc_segment_gather_sum(y, idx3)
  return _tc_final(g_pad, W[:D], p)


# speedup vs baseline: 1.0719x; 1.0041x over previous
import functools

import jax
import jax.numpy as jnp
from jax import lax
from jax.experimental import pallas as pl
from jax.experimental.pallas import tpu as pltpu
from jax.experimental.pallas import tpu_sc as plsc

N_NODES = 10000
DEG = 32
D = 128
LANES = 16
REGS = D // LANES

NC, NS = 2, 16
NW = NC * NS
NPW = 320
NPAD = NW * NPW
CHUNK_NODES = 4
CHUNK_EDGES = CHUNK_NODES * DEG
CHUNKS = NPW // CHUNK_NODES
EPW = NPW * DEG
NBUF = 2
SROWS = 624
N_EDGES = N_NODES * DEG
TAIL = N_EDGES - (NW - 1) * EPW
PADN = EPW - TAIL


def _sc_segment_gather_sum(y, idx, pad_idx):
  mesh = plsc.VectorSubcoreMesh(core_axis_name="c", subcore_axis_name="s")

  @functools.partial(
      pl.kernel,
      out_type=jax.ShapeDtypeStruct((NPAD, D), jnp.float32),
      mesh=mesh,
      scratch_types=[
          pltpu.VMEM((EPW,), jnp.int32),
          pltpu.VMEM((NBUF, CHUNK_EDGES, D), jnp.float32),
          pltpu.VMEM((NBUF, CHUNK_NODES, D), jnp.float32),
          pltpu.VMEM_SHARED((N_NODES, D), jnp.float32),
          [pltpu.SemaphoreType.DMA] * NBUF,
          [pltpu.SemaphoreType.DMA] * NBUF,
      ],
  )
  def body(y_hbm, idx_hbm, pad_hbm, out_hbm, idx_v, gbuf, sbuf, y_sp, gsems,
           ssems):
    wid = lax.axis_index("s") * NC + lax.axis_index("c")
    sid = lax.axis_index("s")
    pltpu.sync_copy(y_hbm.at[pl.ds(sid * SROWS, SROWS)],
                    y_sp.at[pl.ds(sid * SROWS, SROWS)])

    @pl.when(sid == 0)
    def _():
      pltpu.sync_copy(y_hbm.at[pl.ds(NS * SROWS, N_NODES - NS * SROWS)],
                      y_sp.at[pl.ds(NS * SROWS, N_NODES - NS * SROWS)])

    @pl.when(wid < NW - 1)
    def _():
      pltpu.sync_copy(idx_hbm.at[pl.ds(wid * EPW, EPW)], idx_v)

    @pl.when(wid == NW - 1)
    def _():
      pltpu.sync_copy(idx_hbm.at[pl.ds((NW - 1) * EPW, TAIL)],
                      idx_v.at[pl.ds(0, TAIL)])
      pltpu.sync_copy(pad_hbm, idx_v.at[pl.ds(TAIL, PADN)])
    plsc.subcore_barrier()

    def start(chunk, slot):
      pltpu.async_copy(y_sp.at[idx_v.at[pl.ds(chunk * CHUNK_EDGES,
                                              CHUNK_EDGES)]],
                       gbuf.at[slot], gsems[slot])

    def wait(slot):
      pltpu.make_async_copy(y_sp.at[idx_v.at[pl.ds(0, CHUNK_EDGES)]],
                            gbuf.at[slot], gsems[slot]).wait()

    def store(chunk, slot):
      pltpu.async_copy(
          sbuf.at[slot],
          out_hbm.at[pl.ds(wid * NPW + chunk * CHUNK_NODES, CHUNK_NODES)],
          ssems[slot])

    def store_wait(slot):
      pltpu.make_async_copy(
          sbuf.at[slot], out_hbm.at[pl.ds(0, CHUNK_NODES)], ssems[slot]).wait()

    def process(chunk, slot):
      for n in range(CHUNK_NODES):
        base = n * DEG

        def edge_body(e8, accs):
          out = list(accs)
          for de in range(4):
            row = base + e8 * 4 + de
            for r in range(REGS):
              out[r] = out[r] + gbuf[slot, row, pl.ds(r * LANES, LANES)]
          return tuple(out)

        zeros = tuple(jnp.zeros((LANES,), jnp.float32) for _ in range(REGS))
        accs = lax.fori_loop(0, DEG // 4, edge_body, zeros)
        for r in range(REGS):
          sbuf[slot, n, pl.ds(r * LANES, LANES)] = accs[r]
      store(chunk, slot)

    for slot in range(NBUF - 1):
      start(slot, slot)

    def loop_body(g2, carry):
      for b in range(NBUF):
        c = NBUF * g2 + b

        @pl.when(c + NBUF - 1 < CHUNKS)
        def _():
          start(c + NBUF - 1, (b + NBUF - 1) % NBUF)

        wait(b)

        @pl.when(c >= NBUF)
        def _():
          store_wait(b)
        process(c, b)
      return carry

    lax.fori_loop(0, CHUNKS // NBUF, loop_body, 0)
    for slot in range(NBUF):
      store_wait(slot)

  return body(y, idx, pad_idx)


def _tc_partial_kernel(y_ref, w2_ref, b_ref, out_ref):
  acc = jnp.dot(y_ref[...] * jnp.float32(DEG), w2_ref[...],
                preferred_element_type=jnp.float32,
                precision=lax.Precision.HIGHEST)
  out_ref[...] = acc + jnp.float32(DEG) * b_ref[...]


def _tc_partial(y, w2, b2d):
  blk = 2000
  return pl.pallas_call(
      _tc_partial_kernel,
      out_shape=jax.ShapeDtypeStruct((N_NODES, D), jnp.float32),
      grid=(N_NODES // blk,),
      in_specs=[
          pl.BlockSpec((blk, D), lambda i: (i, 0)),
          pl.BlockSpec((D, D), lambda i: (0, 0)),
          pl.BlockSpec((1, D), lambda i: (0, 0)),
      ],
      out_specs=pl.BlockSpec((blk, D), lambda i: (i, 0)),
      compiler_params=pltpu.CompilerParams(
          dimension_semantics=("parallel",)),
  )(y, w2, b2d)


def _tc_final_kernel(g_ref, w1_ref, p_ref, out_ref):
  out_ref[...] = p_ref[...] + jnp.dot(g_ref[...], w1_ref[...],
                                      preferred_element_type=jnp.float32,
                                      precision=lax.Precision.HIGHEST)


def _tc_final(g_pad, w1, p):
  blk = 1000
  return pl.pallas_call(
      _tc_final_kernel,
      out_shape=jax.ShapeDtypeStruct((N_NODES, D), jnp.float32),
      grid=(N_NODES // blk,),
      in_specs=[
          pl.BlockSpec((blk, D), lambda i: (i, 0)),
          pl.BlockSpec((D, D), lambda i: (0, 0)),
          pl.BlockSpec((blk, D), lambda i: (i, 0)),
      ],
      out_specs=pl.BlockSpec((blk, D), lambda i: (i, 0)),
      compiler_params=pltpu.CompilerParams(
          dimension_semantics=("parallel",)),
  )(g_pad, w1, p)


@jax.jit
def kernel(y, neighbors_index, neighbors_row_splits, W, b):
  del neighbors_row_splits
  pad_idx = jnp.arange(PADN, dtype=jnp.int32) % N_NODES
  p = _tc_partial(y, W[D:], b.reshape(1, D))
  g_pad = _sc_segment_gather_sum(y, neighbors_index, pad_idx)
  return _tc_final(g_pad, W[:D], p)
